# baseline (speedup 1.0000x reference)
import functools

import jax
import jax.numpy as jnp
from jax import lax
from jax.experimental import pallas as pl
from jax.experimental.pallas import tpu as pltpu
from jax.experimental.pallas import tpu_sc as plsc

N = 10000
E = 320000
D = 128

N_TILES = 32
CHUNK = 96
N_CHUNKS = 106
E_PER_TILE = CHUNK * N_CHUNKS
E_PAD = N_TILES * E_PER_TILE
N_ACC = 10112
ROWS_PER_TILE = N_ACC // 16
N_PAD_ROWS = N_ACC - N


def _emb_body(a_ref, we_ref, be_ref, out_ref):
    out_ref[...] = (
        jnp.dot(a_ref[...], we_ref[...], preferred_element_type=jnp.float32)
        + be_ref[...]
    )


def _edge_encoder(edge_attr_pad, We_pad, be):
    blk = 1024
    grid = E_PAD // blk
    return pl.pallas_call(
        _emb_body,
        grid=(grid,),
        in_specs=[
            pl.BlockSpec((blk, 8), lambda i: (i, 0)),
            pl.BlockSpec((8, D), lambda i: (0, 0)),
            pl.BlockSpec((1, D), lambda i: (0, 0)),
        ],
        out_specs=pl.BlockSpec((blk, D), lambda i: (i, 0)),
        out_shape=jax.ShapeDtypeStruct((E_PAD, D), jnp.float32),
    )(edge_attr_pad, We_pad, be.reshape(1, D))


def _sc_body(node_hbm, emb_hbm, src_hbm, dst_hbm, out_hbm,
             acc, srci, dsti, xjv, embv, gsems, esems, dsems, ssems, isems):
    cid = lax.axis_index("c")
    sid = lax.axis_index("s")
    wid = cid * 16 + sid

    def _zero_row(r, _):
        for d in range(8):
            embv[0, r, pl.ds(16 * d, 16)] = jnp.zeros((16,), jnp.float32)
        return 0

    lax.fori_loop(0, CHUNK, _zero_row, 0)
    row_base = sid * ROWS_PER_TILE
    for b in range(ROWS_PER_TILE // CHUNK):
        pltpu.sync_copy(embv.at[0], acc.at[pl.ds(row_base + b * CHUNK, CHUNK)])
    pltpu.sync_copy(
        embv.at[0], acc.at[pl.ds(row_base + ROWS_PER_TILE - CHUNK, CHUNK)])
    plsc.subcore_barrier()

    base = wid * E_PER_TILE

    def _fire_srci(g, buf):
        off = pl.multiple_of(base + g * CHUNK, CHUNK)
        pltpu.async_copy(src_hbm.at[pl.ds(off, CHUNK)], srci.at[buf],
                         isems.at[buf])

    def _fire_loads(g, buf):
        off = pl.multiple_of(base + g * CHUNK, CHUNK)
        pltpu.make_async_copy(src_hbm.at[pl.ds(0, CHUNK)], srci.at[buf],
                              isems.at[buf]).wait()
        pltpu.async_copy(node_hbm.at[srci.at[buf]], xjv.at[buf], gsems.at[buf])
        pltpu.async_copy(emb_hbm.at[pl.ds(off, CHUNK)], embv.at[buf],
                         esems.at[buf])
        pltpu.async_copy(dst_hbm.at[pl.ds(off, CHUNK)], dsti.at[buf],
                         dsems.at[buf])

    def _wait_scatter(buf):
        pltpu.make_async_copy(xjv.at[buf], acc.at[dsti.at[buf]],
                              ssems.at[buf]).wait()

    def _half(g, buf):
        nbuf = 1 - buf

        @pl.when(g >= 1)
        def _():
            _wait_scatter(nbuf)

        @pl.when(g < N_CHUNKS - 1)
        def _():
            _fire_loads(g + 1, nbuf)

        pltpu.make_async_copy(node_hbm.at[srci.at[buf]], xjv.at[buf],
                              gsems.at[buf]).wait()
        pltpu.make_async_copy(emb_hbm.at[pl.ds(0, CHUNK)], embv.at[buf],
                              esems.at[buf]).wait()

        @pl.when(g < N_CHUNKS - 2)
        def _():
            _fire_srci(g + 2, buf)

        @plsc.parallel_loop(0, CHUNK, 1, unroll=4)
        def _edge(e):
            for d in range(8):
                s = pl.ds(16 * d, 16)
                xjv[buf, e, s] = jnp.maximum(xjv[buf, e, s] + embv[buf, e, s],
                                             0.0)

        pltpu.make_async_copy(dst_hbm.at[pl.ds(0, CHUNK)], dsti.at[buf],
                              dsems.at[buf]).wait()
        pltpu.async_copy(xjv.at[buf], acc.at[dsti.at[buf]], ssems.at[buf],
                         add=True)

    pltpu.sync_copy(src_hbm.at[pl.ds(base, CHUNK)], srci.at[0])
    pltpu.async_copy(node_hbm.at[srci.at[0]], xjv.at[0], gsems.at[0])
    pltpu.async_copy(emb_hbm.at[pl.ds(base, CHUNK)], embv.at[0], esems.at[0])
    pltpu.async_copy(dst_hbm.at[pl.ds(base, CHUNK)], dsti.at[0], dsems.at[0])
    _fire_srci(1, 1)

    def _pair(k, _):
        _half(2 * k, 0)
        _half(2 * k + 1, 1)
        return 0

    lax.fori_loop(0, N_CHUNKS // 2, _pair, 0)
    _wait_scatter(1)
    plsc.subcore_barrier()

    pltpu.sync_copy(acc.at[pl.ds(row_base, ROWS_PER_TILE)],
                    out_hbm.at[cid, pl.ds(row_base, ROWS_PER_TILE)])


def _sc_scatter(node_embed, emb, src, dst):
    mesh = plsc.VectorSubcoreMesh(core_axis_name="c", subcore_axis_name="s")
    f = pl.kernel(
        _sc_body,
        out_type=jax.ShapeDtypeStruct((2, N_ACC, D), jnp.float32),
        mesh=mesh,
        scratch_types=[
            pltpu.VMEM_SHARED((N_ACC, D), jnp.float32),
            pltpu.VMEM((2, CHUNK), jnp.int32),
            pltpu.VMEM((2, CHUNK), jnp.int32),
            pltpu.VMEM((2, CHUNK, D), jnp.float32),
            pltpu.VMEM((2, CHUNK, D), jnp.float32),
            pltpu.SemaphoreType.DMA((2,)),
            pltpu.SemaphoreType.DMA((2,)),
            pltpu.SemaphoreType.DMA((2,)),
            pltpu.SemaphoreType.DMA((2,)),
            pltpu.SemaphoreType.DMA((2,)),
        ],
    )
    return f(node_embed, emb, src, dst)


def _mlp_body(x_ref, v_ref, eps_ref, w1_ref, b1_ref, g_ref, bt_ref, w2_ref,
              b2_ref, out_ref):
    x = x_ref[...]
    v = v_ref[0, :N, :] + v_ref[1, :N, :]
    h = x * (1.0 + eps_ref[...]) + v
    z1 = jnp.dot(h, w1_ref[...], preferred_element_type=jnp.float32) + b1_ref[...]
    mean = jnp.mean(z1, axis=0, keepdims=True)
    var = jnp.mean((z1 - mean) ** 2, axis=0, keepdims=True)
    z1n = g_ref[...] * ((z1 - mean) / jnp.sqrt(var + 1e-5)) + bt_ref[...]
    z2 = jnp.maximum(z1n, 0.0)
    out_ref[...] = (
        jnp.dot(z2, w2_ref[...], preferred_element_type=jnp.float32) + b2_ref[...]
    )


def _mlp(node_embed, v, eps_param, W1, b1, gamma, beta, W2, b2):
    return pl.pallas_call(
        _mlp_body,
        out_shape=jax.ShapeDtypeStruct((N, D), jnp.float32),
    )(
        node_embed,
        v,
        eps_param.reshape(1, 1),
        W1,
        b1.reshape(1, 2 * D),
        gamma.reshape(1, 2 * D),
        beta.reshape(1, 2 * D),
        W2,
        b2.reshape(1, D),
    )


def kernel(node_embed, edge_index, edge_attr, We, be, W1, b1, gamma, beta, W2,
           b2, eps_param):
    n_pad = E_PAD - E
    src = jnp.pad(edge_index[1].astype(jnp.int32), (0, n_pad))
    pad_dst = N + jnp.arange(n_pad, dtype=jnp.int32) % N_PAD_ROWS
    dst = jnp.concatenate([edge_index[0].astype(jnp.int32), pad_dst])
    ea = jnp.pad(edge_attr, ((0, n_pad), (0, 1)))
    We_pad = jnp.pad(We, ((0, 1), (0, 0)))

    emb = _edge_encoder(ea, We_pad, be)
    v = _sc_scatter(node_embed, emb, src, dst)
    return _mlp(node_embed, v, eps_param, W1, b1, gamma, beta, W2, b2)


# speedup vs baseline: 1.0299x; 1.0299x over previous
import functools

import jax
import jax.numpy as jnp
from jax import lax
from jax.experimental import pallas as pl
from jax.experimental.pallas import tpu as pltpu
from jax.experimental.pallas import tpu_sc as plsc

N = 10000
E = 320000
D = 128

CHUNK = 96
NC0 = 124
NC1 = 88
E_PAIR = CHUNK * (NC0 + NC1)
E_PAD = 16 * E_PAIR
CORE1_BASE = 16 * NC0 * CHUNK
N_ACC = 10112
ROWS_PER_TILE = N_ACC // 16
N_PAD_ROWS = N_ACC - N


def _emb_body(a_ref, we_ref, be_ref, out_ref):
    out_ref[...] = (
        jnp.dot(a_ref[...], we_ref[...], preferred_element_type=jnp.float32)
        + be_ref[...]
    )


def _edge_encoder(edge_attr_pad, We_pad, be):
    blk = 1024
    grid = E_PAD // blk
    return pl.pallas_call(
        _emb_body,
        grid=(grid,),
        in_specs=[
            pl.BlockSpec((blk, 8), lambda i: (i, 0)),
            pl.BlockSpec((8, D), lambda i: (0, 0)),
            pl.BlockSpec((1, D), lambda i: (0, 0)),
        ],
        out_specs=pl.BlockSpec((blk, D), lambda i: (i, 0)),
        out_shape=jax.ShapeDtypeStruct((E_PAD, D), jnp.float32),
    )(edge_attr_pad, We_pad, be.reshape(1, D))


def _sc_body(node_hbm, emb_hbm, src_hbm, dst_hbm, out_hbm,
             acc, srci, dsti, xjv, embv, gsems, esems, dsems, ssems, isems):
    cid = lax.axis_index("c")
    sid = lax.axis_index("s")

    def _zero_row(r, _):
        for d in range(8):
            embv[0, r, pl.ds(16 * d, 16)] = jnp.zeros((16,), jnp.float32)
        return 0

    lax.fori_loop(0, CHUNK, _zero_row, 0)
    row_base = sid * ROWS_PER_TILE
    for b in range(ROWS_PER_TILE // CHUNK):
        pltpu.sync_copy(embv.at[0], acc.at[pl.ds(row_base + b * CHUNK, CHUNK)])
    pltpu.sync_copy(
        embv.at[0], acc.at[pl.ds(row_base + ROWS_PER_TILE - CHUNK, CHUNK)])
    plsc.subcore_barrier()

    n_chunks = jnp.where(cid == 0, NC0, NC1)
    base = jnp.where(cid == 0, sid * (NC0 * CHUNK),
                     CORE1_BASE + sid * (NC1 * CHUNK))

    def _fire_srci(g, buf):
        off = pl.multiple_of(base + g * CHUNK, CHUNK)
        pltpu.async_copy(src_hbm.at[pl.ds(off, CHUNK)], srci.at[buf],
                         isems.at[buf])

    def _fire_loads(g, buf):
        off = pl.multiple_of(base + g * CHUNK, CHUNK)
        pltpu.make_async_copy(src_hbm.at[pl.ds(0, CHUNK)], srci.at[buf],
                              isems.at[buf]).wait()
        pltpu.async_copy(node_hbm.at[srci.at[buf]], xjv.at[buf], gsems.at[buf])
        pltpu.async_copy(emb_hbm.at[pl.ds(off, CHUNK)], embv.at[buf],
                         esems.at[buf])
        pltpu.async_copy(dst_hbm.at[pl.ds(off, CHUNK)], dsti.at[buf],
                         dsems.at[buf])

    def _wait_scatter(buf):
        pltpu.make_async_copy(xjv.at[buf], acc.at[dsti.at[buf]],
                              ssems.at[buf]).wait()

    def _half(g, buf):
        nbuf = 1 - buf

        @pl.when(g >= 1)
        def _():
            _wait_scatter(nbuf)

        @pl.when(g < n_chunks - 1)
        def _():
            _fire_loads(g + 1, nbuf)

        pltpu.make_async_copy(node_hbm.at[srci.at[buf]], xjv.at[buf],
                              gsems.at[buf]).wait()
        pltpu.make_async_copy(emb_hbm.at[pl.ds(0, CHUNK)], embv.at[buf],
                              esems.at[buf]).wait()

        @pl.when(g < n_chunks - 2)
        def _():
            _fire_srci(g + 2, buf)

        @plsc.parallel_loop(0, CHUNK, 1, unroll=2)
        def _edge(e):
            for d in range(8):
                s = pl.ds(16 * d, 16)
                xjv[buf, e, s] = jnp.maximum(xjv[buf, e, s] + embv[buf, e, s],
                                             0.0)

        pltpu.make_async_copy(dst_hbm.at[pl.ds(0, CHUNK)], dsti.at[buf],
                              dsems.at[buf]).wait()
        pltpu.async_copy(xjv.at[buf], acc.at[dsti.at[buf]], ssems.at[buf],
                         add=True)

    pltpu.sync_copy(src_hbm.at[pl.ds(pl.multiple_of(base, CHUNK), CHUNK)],
                    srci.at[0])
    pltpu.async_copy(node_hbm.at[srci.at[0]], xjv.at[0], gsems.at[0])
    pltpu.async_copy(emb_hbm.at[pl.ds(pl.multiple_of(base, CHUNK), CHUNK)],
                     embv.at[0], esems.at[0])
    pltpu.async_copy(dst_hbm.at[pl.ds(pl.multiple_of(base, CHUNK), CHUNK)],
                     dsti.at[0], dsems.at[0])
    _fire_srci(1, 1)

    def _pair(k, _):
        _half(2 * k, 0)
        _half(2 * k + 1, 1)
        return 0

    lax.fori_loop(0, n_chunks // 2, _pair, 0)
    _wait_scatter(1)
    plsc.subcore_barrier()

    pltpu.sync_copy(acc.at[pl.ds(row_base, ROWS_PER_TILE)],
                    out_hbm.at[cid, pl.ds(row_base, ROWS_PER_TILE)])


def _sc_scatter(node_embed, emb, src, dst):
    mesh = plsc.VectorSubcoreMesh(core_axis_name="c", subcore_axis_name="s")
    f = pl.kernel(
        _sc_body,
        out_type=jax.ShapeDtypeStruct((2, N_ACC, D), jnp.float32),
        mesh=mesh,
        scratch_types=[
            pltpu.VMEM_SHARED((N_ACC, D), jnp.float32),
            pltpu.VMEM((2, CHUNK), jnp.int32),
            pltpu.VMEM((2, CHUNK), jnp.int32),
            pltpu.VMEM((2, CHUNK, D), jnp.float32),
            pltpu.VMEM((2, CHUNK, D), jnp.float32),
            pltpu.SemaphoreType.DMA((2,)),
            pltpu.SemaphoreType.DMA((2,)),
            pltpu.SemaphoreType.DMA((2,)),
            pltpu.SemaphoreType.DMA((2,)),
            pltpu.SemaphoreType.DMA((2,)),
        ],
    )
    return f(node_embed, emb, src, dst)


def _mlp_body(x_ref, v_ref, eps_ref, w1_ref, b1_ref, g_ref, bt_ref, w2_ref,
              b2_ref, out_ref):
    x = x_ref[...]
    v = v_ref[0, :N, :] + v_ref[1, :N, :]
    h = x * (1.0 + eps_ref[...]) + v
    z1 = jnp.dot(h, w1_ref[...], preferred_element_type=jnp.float32) + b1_ref[...]
    mean = jnp.mean(z1, axis=0, keepdims=True)
    var = jnp.mean((z1 - mean) ** 2, axis=0, keepdims=True)
    z1n = g_ref[...] * ((z1 - mean) / jnp.sqrt(var + 1e-5)) + bt_ref[...]
    z2 = jnp.maximum(z1n, 0.0)
    out_ref[...] = (
        jnp.dot(z2, w2_ref[...], preferred_element_type=jnp.float32) + b2_ref[...]
    )


def _mlp(node_embed, v, eps_param, W1, b1, gamma, beta, W2, b2):
    return pl.pallas_call(
        _mlp_body,
        out_shape=jax.ShapeDtypeStruct((N, D), jnp.float32),
    )(
        node_embed,
        v,
        eps_param.reshape(1, 1),
        W1,
        b1.reshape(1, 2 * D),
        gamma.reshape(1, 2 * D),
        beta.reshape(1, 2 * D),
        W2,
        b2.reshape(1, D),
    )


def kernel(node_embed, edge_index, edge_attr, We, be, W1, b1, gamma, beta, W2,
           b2, eps_param):
    n_pad = E_PAD - E
    src = jnp.pad(edge_index[1].astype(jnp.int32), (0, n_pad))
    pad_dst = N + jnp.arange(n_pad, dtype=jnp.int32) % N_PAD_ROWS
    dst = jnp.concatenate([edge_index[0].astype(jnp.int32), pad_dst])
    ea = jnp.pad(edge_attr, ((0, n_pad), (0, 1)))
    We_pad = jnp.pad(We, ((0, 1), (0, 0)))

    emb = _edge_encoder(ea, We_pad, be)
    v = _sc_scatter(node_embed, emb, src, dst)
    return _mlp(node_embed, v, eps_param, W1, b1, gamma, beta, W2, b2)


# speedup vs baseline: 1.0689x; 1.0379x over previous
import functools

import jax
import jax.numpy as jnp
from jax import lax
from jax.experimental import pallas as pl
from jax.experimental.pallas import tpu as pltpu
from jax.experimental.pallas import tpu_sc as plsc

N = 10000
E = 320000
D = 128

CHUNK = 96
NC0 = 144
NC1 = 68
E_PAIR = CHUNK * (NC0 + NC1)
E_PAD = 16 * E_PAIR
CORE1_BASE = 16 * NC0 * CHUNK
N_ACC = 10112
ROWS_PER_TILE = N_ACC // 16
N_PAD_ROWS = N_ACC - N


def _emb_body(a_ref, we_ref, be_ref, out_ref):
    out_ref[...] = (
        jnp.dot(a_ref[...], we_ref[...], preferred_element_type=jnp.float32)
        + be_ref[...]
    )


def _edge_encoder(edge_attr_pad, We_pad, be):
    blk = 1024
    grid = E_PAD // blk
    return pl.pallas_call(
        _emb_body,
        grid=(grid,),
        in_specs=[
            pl.BlockSpec((blk, 8), lambda i: (i, 0)),
            pl.BlockSpec((8, D), lambda i: (0, 0)),
            pl.BlockSpec((1, D), lambda i: (0, 0)),
        ],
        out_specs=pl.BlockSpec((blk, D), lambda i: (i, 0)),
        out_shape=jax.ShapeDtypeStruct((E_PAD, D), jnp.float32),
    )(edge_attr_pad, We_pad, be.reshape(1, D))


def _sc_body(node_hbm, emb_hbm, src_hbm, dst_hbm, out_hbm,
             acc, srci, dsti, xjv, embv, gsems, esems, dsems, ssems, isems):
    cid = lax.axis_index("c")
    sid = lax.axis_index("s")

    def _zero_row(r, _):
        for d in range(8):
            embv[0, r, pl.ds(16 * d, 16)] = jnp.zeros((16,), jnp.float32)
        return 0

    lax.fori_loop(0, CHUNK, _zero_row, 0)
    row_base = sid * ROWS_PER_TILE
    for b in range(ROWS_PER_TILE // CHUNK):
        pltpu.sync_copy(embv.at[0], acc.at[pl.ds(row_base + b * CHUNK, CHUNK)])
    pltpu.sync_copy(
        embv.at[0], acc.at[pl.ds(row_base + ROWS_PER_TILE - CHUNK, CHUNK)])
    plsc.subcore_barrier()

    n_chunks = jnp.where(cid == 0, NC0, NC1)
    base = jnp.where(cid == 0, sid * (NC0 * CHUNK),
                     CORE1_BASE + sid * (NC1 * CHUNK))

    def _fire_srci(g, buf):
        off = pl.multiple_of(base + g * CHUNK, CHUNK)
        pltpu.async_copy(src_hbm.at[pl.ds(off, CHUNK)], srci.at[buf],
                         isems.at[buf])

    def _fire_loads(g, buf):
        off = pl.multiple_of(base + g * CHUNK, CHUNK)
        pltpu.make_async_copy(src_hbm.at[pl.ds(0, CHUNK)], srci.at[buf],
                              isems.at[buf]).wait()
        pltpu.async_copy(node_hbm.at[srci.at[buf]], xjv.at[buf], gsems.at[buf])
        pltpu.async_copy(emb_hbm.at[pl.ds(off, CHUNK)], embv.at[buf],
                         esems.at[buf])
        pltpu.async_copy(dst_hbm.at[pl.ds(off, CHUNK)], dsti.at[buf],
                         dsems.at[buf])

    def _wait_scatter(buf):
        pltpu.make_async_copy(xjv.at[buf], acc.at[dsti.at[buf]],
                              ssems.at[buf]).wait()

    def _half(g, buf):
        nbuf = 1 - buf

        @pl.when(g >= 1)
        def _():
            _wait_scatter(nbuf)

        @pl.when(g < n_chunks - 1)
        def _():
            _fire_loads(g + 1, nbuf)

        pltpu.make_async_copy(node_hbm.at[srci.at[buf]], xjv.at[buf],
                              gsems.at[buf]).wait()
        pltpu.make_async_copy(emb_hbm.at[pl.ds(0, CHUNK)], embv.at[buf],
                              esems.at[buf]).wait()

        @pl.when(g < n_chunks - 2)
        def _():
            _fire_srci(g + 2, buf)

        @plsc.parallel_loop(0, CHUNK, 1, unroll=2)
        def _edge(e):
            for d in range(8):
                s = pl.ds(16 * d, 16)
                xjv[buf, e, s] = jnp.maximum(xjv[buf, e, s] + embv[buf, e, s],
                                             0.0)

        pltpu.make_async_copy(dst_hbm.at[pl.ds(0, CHUNK)], dsti.at[buf],
                              dsems.at[buf]).wait()
        pltpu.async_copy(xjv.at[buf], acc.at[dsti.at[buf]], ssems.at[buf],
                         add=True)

    pltpu.sync_copy(src_hbm.at[pl.ds(pl.multiple_of(base, CHUNK), CHUNK)],
                    srci.at[0])
    pltpu.async_copy(node_hbm.at[srci.at[0]], xjv.at[0], gsems.at[0])
    pltpu.async_copy(emb_hbm.at[pl.ds(pl.multiple_of(base, CHUNK), CHUNK)],
                     embv.at[0], esems.at[0])
    pltpu.async_copy(dst_hbm.at[pl.ds(pl.multiple_of(base, CHUNK), CHUNK)],
                     dsti.at[0], dsems.at[0])
    _fire_srci(1, 1)

    def _pair(k, _):
        _half(2 * k, 0)
        _half(2 * k + 1, 1)
        return 0

    lax.fori_loop(0, n_chunks // 2, _pair, 0)
    _wait_scatter(1)
    plsc.subcore_barrier()

    pltpu.sync_copy(acc.at[pl.ds(row_base, ROWS_PER_TILE)],
                    out_hbm.at[cid, pl.ds(row_base, ROWS_PER_TILE)])


def _sc_scatter(node_embed, emb, src, dst):
    mesh = plsc.VectorSubcoreMesh(core_axis_name="c", subcore_axis_name="s")
    f = pl.kernel(
        _sc_body,
        out_type=jax.ShapeDtypeStruct((2, N_ACC, D), jnp.float32),
        mesh=mesh,
        scratch_types=[
            pltpu.VMEM_SHARED((N_ACC, D), jnp.float32),
            pltpu.VMEM((2, CHUNK), jnp.int32),
            pltpu.VMEM((2, CHUNK), jnp.int32),
            pltpu.VMEM((2, CHUNK, D), jnp.float32),
            pltpu.VMEM((2, CHUNK, D), jnp.float32),
            pltpu.SemaphoreType.DMA((2,)),
            pltpu.SemaphoreType.DMA((2,)),
            pltpu.SemaphoreType.DMA((2,)),
            pltpu.SemaphoreType.DMA((2,)),
            pltpu.SemaphoreType.DMA((2,)),
        ],
    )
    return f(node_embed, emb, src, dst)


def _mlp_body(x_ref, v_ref, eps_ref, w1_ref, b1_ref, g_ref, bt_ref, w2_ref,
              b2_ref, out_ref):
    x = x_ref[...]
    v = v_ref[0, :N, :] + v_ref[1, :N, :]
    h = x * (1.0 + eps_ref[...]) + v
    z1 = jnp.dot(h, w1_ref[...], preferred_element_type=jnp.float32) + b1_ref[...]
    mean = jnp.mean(z1, axis=0, keepdims=True)
    var = jnp.mean((z1 - mean) ** 2, axis=0, keepdims=True)
    z1n = g_ref[...] * ((z1 - mean) / jnp.sqrt(var + 1e-5)) + bt_ref[...]
    z2 = jnp.maximum(z1n, 0.0)
    out_ref[...] = (
        jnp.dot(z2, w2_ref[...], preferred_element_type=jnp.float32) + b2_ref[...]
    )


def _mlp(node_embed, v, eps_param, W1, b1, gamma, beta, W2, b2):
    return pl.pallas_call(
        _mlp_body,
        out_shape=jax.ShapeDtypeStruct((N, D), jnp.float32),
    )(
        node_embed,
        v,
        eps_param.reshape(1, 1),
        W1,
        b1.reshape(1, 2 * D),
        gamma.reshape(1, 2 * D),
        beta.reshape(1, 2 * D),
        W2,
        b2.reshape(1, D),
    )


def kernel(node_embed, edge_index, edge_attr, We, be, W1, b1, gamma, beta, W2,
           b2, eps_param):
    n_pad = E_PAD - E
    src = jnp.pad(edge_index[1].astype(jnp.int32), (0, n_pad))
    pad_dst = N + jnp.arange(n_pad, dtype=jnp.int32) % N_PAD_ROWS
    dst = jnp.concatenate([edge_index[0].astype(jnp.int32), pad_dst])
    ea = jnp.pad(edge_attr, ((0, n_pad), (0, 1)))
    We_pad = jnp.pad(We, ((0, 1), (0, 0)))

    emb = _edge_encoder(ea, We_pad, be)
    v = _sc_scatter(node_embed, emb, src, dst)
    return _mlp(node_embed, v, eps_param, W1, b1, gamma, beta, W2, b2)


# speedup vs baseline: 1.0882x; 1.0181x over previous
import functools

import jax
import jax.numpy as jnp
from jax import lax
from jax.experimental import pallas as pl
from jax.experimental.pallas import tpu as pltpu
from jax.experimental.pallas import tpu_sc as plsc

N = 10000
E = 320000
D = 128

CHUNK = 96
NC0 = 156
NC1 = 56
E_PAIR = CHUNK * (NC0 + NC1)
E_PAD = 16 * E_PAIR
CORE1_BASE = 16 * NC0 * CHUNK
N_ACC = 10112
ROWS_PER_TILE = N_ACC // 16
N_PAD_ROWS = N_ACC - N


def _emb_body(a_ref, we_ref, be_ref, out_ref):
    out_ref[...] = (
        jnp.dot(a_ref[...], we_ref[...], preferred_element_type=jnp.float32)
        + be_ref[...]
    )


def _edge_encoder(edge_attr_pad, We_pad, be):
    blk = 1024
    grid = E_PAD // blk
    return pl.pallas_call(
        _emb_body,
        grid=(grid,),
        in_specs=[
            pl.BlockSpec((blk, 8), lambda i: (i, 0)),
            pl.BlockSpec((8, D), lambda i: (0, 0)),
            pl.BlockSpec((1, D), lambda i: (0, 0)),
        ],
        out_specs=pl.BlockSpec((blk, D), lambda i: (i, 0)),
        out_shape=jax.ShapeDtypeStruct((E_PAD, D), jnp.float32),
    )(edge_attr_pad, We_pad, be.reshape(1, D))


def _sc_body(node_hbm, emb_hbm, src_hbm, dst_hbm, out_hbm,
             acc, srci, dsti, xjv, embv, gsems, esems, dsems, ssems, isems):
    cid = lax.axis_index("c")
    sid = lax.axis_index("s")

    def _zero_row(r, _):
        for d in range(8):
            embv[0, r, pl.ds(16 * d, 16)] = jnp.zeros((16,), jnp.float32)
        return 0

    lax.fori_loop(0, CHUNK, _zero_row, 0)
    row_base = sid * ROWS_PER_TILE
    for b in range(ROWS_PER_TILE // CHUNK):
        pltpu.sync_copy(embv.at[0], acc.at[pl.ds(row_base + b * CHUNK, CHUNK)])
    pltpu.sync_copy(
        embv.at[0], acc.at[pl.ds(row_base + ROWS_PER_TILE - CHUNK, CHUNK)])
    plsc.subcore_barrier()

    n_chunks = jnp.where(cid == 0, NC0, NC1)
    base = jnp.where(cid == 0, sid * (NC0 * CHUNK),
                     CORE1_BASE + sid * (NC1 * CHUNK))

    def _fire_srci(g, buf):
        off = pl.multiple_of(base + g * CHUNK, CHUNK)
        pltpu.async_copy(src_hbm.at[pl.ds(off, CHUNK)], srci.at[buf],
                         isems.at[buf])

    def _fire_loads(g, buf):
        off = pl.multiple_of(base + g * CHUNK, CHUNK)
        pltpu.make_async_copy(src_hbm.at[pl.ds(0, CHUNK)], srci.at[buf],
                              isems.at[buf]).wait()
        pltpu.async_copy(node_hbm.at[srci.at[buf]], xjv.at[buf], gsems.at[buf])
        pltpu.async_copy(emb_hbm.at[pl.ds(off, CHUNK)], embv.at[buf],
                         esems.at[buf])
        pltpu.async_copy(dst_hbm.at[pl.ds(off, CHUNK)], dsti.at[buf],
                         dsems.at[buf])

    def _wait_scatter(buf):
        pltpu.make_async_copy(xjv.at[buf], acc.at[dsti.at[buf]],
                              ssems.at[buf]).wait()

    def _half(g, buf):
        nbuf = 1 - buf

        @pl.when(g >= 1)
        def _():
            _wait_scatter(nbuf)

        @pl.when(g < n_chunks - 1)
        def _():
            _fire_loads(g + 1, nbuf)

        pltpu.make_async_copy(node_hbm.at[srci.at[buf]], xjv.at[buf],
                              gsems.at[buf]).wait()
        pltpu.make_async_copy(emb_hbm.at[pl.ds(0, CHUNK)], embv.at[buf],
                              esems.at[buf]).wait()

        @pl.when(g < n_chunks - 2)
        def _():
            _fire_srci(g + 2, buf)

        @plsc.parallel_loop(0, CHUNK, 1, unroll=2)
        def _edge(e):
            for d in range(8):
                s = pl.ds(16 * d, 16)
                xjv[buf, e, s] = jnp.maximum(xjv[buf, e, s] + embv[buf, e, s],
                                             0.0)

        pltpu.make_async_copy(dst_hbm.at[pl.ds(0, CHUNK)], dsti.at[buf],
                              dsems.at[buf]).wait()
        pltpu.async_copy(xjv.at[buf], acc.at[dsti.at[buf]], ssems.at[buf],
                         add=True)

    pltpu.sync_copy(src_hbm.at[pl.ds(pl.multiple_of(base, CHUNK), CHUNK)],
                    srci.at[0])
    pltpu.async_copy(node_hbm.at[srci.at[0]], xjv.at[0], gsems.at[0])
    pltpu.async_copy(emb_hbm.at[pl.ds(pl.multiple_of(base, CHUNK), CHUNK)],
                     embv.at[0], esems.at[0])
    pltpu.async_copy(dst_hbm.at[pl.ds(pl.multiple_of(base, CHUNK), CHUNK)],
                     dsti.at[0], dsems.at[0])
    _fire_srci(1, 1)

    def _pair(k, _):
        _half(2 * k, 0)
        _half(2 * k + 1, 1)
        return 0

    lax.fori_loop(0, n_chunks // 2, _pair, 0)
    _wait_scatter(1)
    plsc.subcore_barrier()

    pltpu.sync_copy(acc.at[pl.ds(row_base, ROWS_PER_TILE)],
                    out_hbm.at[cid, pl.ds(row_base, ROWS_PER_TILE)])


def _sc_scatter(node_embed, emb, src, dst):
    mesh = plsc.VectorSubcoreMesh(core_axis_name="c", subcore_axis_name="s")
    f = pl.kernel(
        _sc_body,
        out_type=jax.ShapeDtypeStruct((2, N_ACC, D), jnp.float32),
        mesh=mesh,
        scratch_types=[
            pltpu.VMEM_SHARED((N_ACC, D), jnp.float32),
            pltpu.VMEM((2, CHUNK), jnp.int32),
            pltpu.VMEM((2, CHUNK), jnp.int32),
            pltpu.VMEM((2, CHUNK, D), jnp.float32),
            pltpu.VMEM((2, CHUNK, D), jnp.float32),
            pltpu.SemaphoreType.DMA((2,)),
            pltpu.SemaphoreType.DMA((2,)),
            pltpu.SemaphoreType.DMA((2,)),
            pltpu.SemaphoreType.DMA((2,)),
            pltpu.SemaphoreType.DMA((2,)),
        ],
    )
    return f(node_embed, emb, src, dst)


def _mlp_body(x_ref, v_ref, eps_ref, w1_ref, b1_ref, g_ref, bt_ref, w2_ref,
              b2_ref, out_ref):
    x = x_ref[...]
    v = v_ref[0, :N, :] + v_ref[1, :N, :]
    h = x * (1.0 + eps_ref[...]) + v
    z1 = jnp.dot(h, w1_ref[...], preferred_element_type=jnp.float32) + b1_ref[...]
    mean = jnp.mean(z1, axis=0, keepdims=True)
    var = jnp.mean((z1 - mean) ** 2, axis=0, keepdims=True)
    z1n = g_ref[...] * ((z1 - mean) / jnp.sqrt(var + 1e-5)) + bt_ref[...]
    z2 = jnp.maximum(z1n, 0.0)
    out_ref[...] = (
        jnp.dot(z2, w2_ref[...], preferred_element_type=jnp.float32) + b2_ref[...]
    )


def _mlp(node_embed, v, eps_param, W1, b1, gamma, beta, W2, b2):
    return pl.pallas_call(
        _mlp_body,
        out_shape=jax.ShapeDtypeStruct((N, D), jnp.float32),
    )(
        node_embed,
        v,
        eps_param.reshape(1, 1),
        W1,
        b1.reshape(1, 2 * D),
        gamma.reshape(1, 2 * D),
        beta.reshape(1, 2 * D),
        W2,
        b2.reshape(1, D),
    )


def kernel(node_embed, edge_index, edge_attr, We, be, W1, b1, gamma, beta, W2,
           b2, eps_param):
    n_pad = E_PAD - E
    src = jnp.pad(edge_index[1].astype(jnp.int32), (0, n_pad))
    pad_dst = N + jnp.arange(n_pad, dtype=jnp.int32) % N_PAD_ROWS
    dst = jnp.concatenate([edge_index[0].astype(jnp.int32), pad_dst])
    ea = jnp.pad(edge_attr, ((0, n_pad), (0, 1)))
    We_pad = jnp.pad(We, ((0, 1), (0, 0)))

    emb = _edge_encoder(ea, We_pad, be)
    v = _sc_scatter(node_embed, emb, src, dst)
    return _mlp(node_embed, v, eps_param, W1, b1, gamma, beta, W2, b2)
